# baseline (speedup 1.0000x reference)
import functools

import jax
import jax.numpy as jnp
from jax import lax
from jax.experimental import pallas as pl
from jax.experimental.pallas import tpu as pltpu
from jax.experimental.pallas import tpu_sc as plsc

_D = 768
_MAX_REL = 32
_S = 512


def _clip_idx(p):
    return jnp.clip(511 - p, -_MAX_REL, _MAX_REL) + _MAX_REL


def _rpe_sc_kernel(table_hbm, out_hbm, idx_v, arena_a, arena_c, gsem, osem):
    nc = 2
    cid = lax.axis_index("c")
    sid = lax.axis_index("s")
    lane = lax.iota(jnp.int32, 16)
    wid = sid * nc + cid

    pair = lax.shift_right_logical(wid, 1)
    odd = wid & 1
    task_a = 3 * pair + odd
    task_c = 3 * pair + 2
    r_a, strip_a = task_a & 7, lax.shift_right_logical(task_a, 3)
    r_c, strip_c = task_c & 7, lax.shift_right_logical(task_c, 3)

    def build(arena_v, r, strip):
        col = 128 * strip
        for u in range(16):
            idx_v[pl.ds(0, 16)] = _clip_idx(384 + r + 16 * u + lane)
            pltpu.async_copy(
                table_hbm.at[idx_v, pl.ds(col, 128)],
                arena_v.at[pl.ds(16 * u, 16)],
                gsem,
            ).wait()
        idx_v[pl.ds(0, 16)] = 64 + 0 * lane
        for u in range(4):
            pltpu.async_copy(
                table_hbm.at[idx_v, pl.ds(col, 128)],
                arena_v.at[pl.ds(256 + 16 * u, 16)],
                gsem,
            ).wait()
        idx_v[pl.ds(0, 16)] = 0 * lane
        for u in range(4):
            pltpu.async_copy(
                table_hbm.at[idx_v, pl.ds(col, 128)],
                arena_v.at[pl.ds(320 + 16 * u, 16)],
                gsem,
            ).wait()

    build(arena_a, r_a, strip_a)
    build(arena_c, r_c, strip_c)

    def plan(i, r, strip, arena_v):
        s = 511 - i
        a = lax.shift_right_arithmetic(s, 3)
        col = 128 * strip
        out = []
        for e in range(8):
            w0 = s + 64 * e
            g = jnp.where(
                w0 >= 544, 40, jnp.where(w0 <= 416, 32, a + 8 * e - 48)
            )
            out.append((
                arena_v.at[pl.ds(8 * g, 64)],
                out_hbm.at[i, pl.ds(64 * e, 64), pl.ds(col, 128)],
            ))
        return out

    def fire(p):
        for src, dst in p:
            pltpu.async_copy(src, dst, osem)

    def drain(p):
        for src, dst in p:
            pltpu.make_async_copy(src, dst, osem).wait()

    def slab_i(n, r):
        return 8 * n + 7 - r

    def body_a(n, carry):
        fire(plan(slab_i(n, r_a), r_a, strip_a, arena_a))

        @pl.when(n > 0)
        def _():
            drain(plan(slab_i(n - 1, r_a), r_a, strip_a, arena_a))

        return carry

    lax.fori_loop(0, 64, body_a, 0)
    drain(plan(slab_i(63, r_a), r_a, strip_a, arena_a))

    n0 = 32 * odd

    def body_c(n, carry):
        fire(plan(slab_i(n0 + n, r_c), r_c, strip_c, arena_c))

        @pl.when(n > 0)
        def _():
            drain(plan(slab_i(n0 + n - 1, r_c), r_c, strip_c, arena_c))

        return carry

    lax.fori_loop(0, 32, body_c, 0)
    drain(plan(slab_i(n0 + 31, r_c), r_c, strip_c, arena_c))


def kernel(table, seq_len):
    del seq_len
    mesh = plsc.VectorSubcoreMesh(core_axis_name="c", subcore_axis_name="s")
    k = functools.partial(
        pl.kernel,
        mesh=mesh,
        out_type=jax.ShapeDtypeStruct((_S, _S, _D), jnp.float32),
        scratch_types=[
            pltpu.VMEM((16,), jnp.int32),
            pltpu.VMEM((384, 128), jnp.float32),
            pltpu.VMEM((384, 128), jnp.float32),
            pltpu.SemaphoreType.DMA,
            pltpu.SemaphoreType.DMA,
        ],
    )(_rpe_sc_kernel)
    return k(table)


# speedup vs baseline: 1.1363x; 1.1363x over previous
import functools

import jax
import jax.numpy as jnp
from jax import lax
from jax.experimental import pallas as pl
from jax.experimental.pallas import tpu as pltpu
from jax.experimental.pallas import tpu_sc as plsc

_D = 768
_MAX_REL = 32
_S = 512
_AROWS = 144


def _clip_idx(p):
    return jnp.clip(511 - p, -_MAX_REL, _MAX_REL) + _MAX_REL


def _rpe_sc_kernel(table_hbm, out_hbm, idx_v, arena_v, gsem, osem):
    nc = 2
    cid = lax.axis_index("c")
    sid = lax.axis_index("s")
    lane = lax.iota(jnp.int32, 16)
    wid = sid * nc + cid
    r = wid & 7
    q = lax.shift_right_logical(wid, 3)

    for u in range(_AROWS // 16):
        idx_v[pl.ds(0, 16)] = _clip_idx(440 + r + 16 * u + lane)
        pltpu.async_copy(
            table_hbm.at[idx_v], arena_v.at[pl.ds(16 * u, 16)], gsem
        ).wait()

    def plan(i):
        s = 511 - i
        a = lax.shift_right_arithmetic(s, 3)
        out = []
        for e in range(16):
            w0 = s + 32 * e
            g = jnp.where(
                w0 >= 544, 13, jnp.where(w0 <= 448, 0, a + 4 * e - 55)
            )
            out.append((
                arena_v.at[pl.ds(8 * g, 32)],
                out_hbm.at[i, pl.ds(32 * e, 32)],
            ))
        return out

    def fire(p):
        for src, dst in p:
            pltpu.async_copy(src, dst, osem)

    def drain(p):
        for src, dst in p:
            pltpu.make_async_copy(src, dst, osem).wait()

    def slab_i(n):
        return 8 * n + 7 - r

    def body(t, carry):
        fire(plan(slab_i(16 * q + t)))

        @pl.when(t > 0)
        def _():
            drain(plan(slab_i(16 * q + t - 1)))

        return carry

    lax.fori_loop(0, 16, body, 0)
    drain(plan(slab_i(16 * q + 15)))


def kernel(table, seq_len):
    del seq_len
    mesh = plsc.VectorSubcoreMesh(core_axis_name="c", subcore_axis_name="s")
    k = functools.partial(
        pl.kernel,
        mesh=mesh,
        out_type=jax.ShapeDtypeStruct((_S, _S, _D), jnp.float32),
        scratch_types=[
            pltpu.VMEM((16,), jnp.int32),
            pltpu.VMEM((_AROWS, _D), jnp.float32),
            pltpu.SemaphoreType.DMA,
            pltpu.SemaphoreType.DMA,
        ],
    )(_rpe_sc_kernel)
    return k(table)


# speedup vs baseline: 1.1378x; 1.0013x over previous
import functools

import jax
import jax.numpy as jnp
from jax import lax
from jax.experimental import pallas as pl
from jax.experimental.pallas import tpu as pltpu
from jax.experimental.pallas import tpu_sc as plsc

_D = 768
_MAX_REL = 32
_S = 512
_AROWS = 144


def _clip_idx(p):
    return jnp.clip(511 - p, -_MAX_REL, _MAX_REL) + _MAX_REL


def _rpe_sc_kernel(table_hbm, out_hbm, idx_v, arena_v, gsem, osem):
    nc = 2
    cid = lax.axis_index("c")
    sid = lax.axis_index("s")
    lane = lax.iota(jnp.int32, 16)
    wid = sid * nc + cid
    r = wid & 7
    q = lax.shift_right_logical(wid, 3)

    for u in range(_AROWS // 16):
        idx_v[pl.ds(0, 16)] = _clip_idx(440 + r + 16 * u + lane)
        pltpu.async_copy(
            table_hbm.at[idx_v], arena_v.at[pl.ds(16 * u, 16)], gsem
        ).wait()

    def plan(i):
        s = 511 - i
        a = lax.shift_right_arithmetic(s, 3)
        out = []
        for e in range(16):
            w0 = s + 32 * e
            g = jnp.where(
                w0 >= 544, 13, jnp.where(w0 <= 448, 0, a + 4 * e - 55)
            )
            out.append((
                arena_v.at[pl.ds(8 * g, 32)],
                out_hbm.at[i, pl.ds(32 * e, 32)],
            ))
        return out

    def fire(p):
        for src, dst in p:
            pltpu.async_copy(src, dst, osem)

    def drain_slab(i):
        pltpu.make_async_copy(out_hbm.at[i], out_hbm.at[i], osem).wait()

    def slab_i(n):
        return 8 * n + 7 - r

    for t in range(16):
        fire(plan(slab_i(16 * q + t)))
        if t >= 2:
            drain_slab(slab_i(16 * q + t - 2))
    drain_slab(slab_i(16 * q + 14))
    drain_slab(slab_i(16 * q + 15))


def kernel(table, seq_len):
    del seq_len
    mesh = plsc.VectorSubcoreMesh(core_axis_name="c", subcore_axis_name="s")
    k = functools.partial(
        pl.kernel,
        mesh=mesh,
        out_type=jax.ShapeDtypeStruct((_S, _S, _D), jnp.float32),
        scratch_types=[
            pltpu.VMEM((16,), jnp.int32),
            pltpu.VMEM((_AROWS, _D), jnp.float32),
            pltpu.SemaphoreType.DMA,
            pltpu.SemaphoreType.DMA,
        ],
    )(_rpe_sc_kernel)
    return k(table)


# speedup vs baseline: 1.4610x; 1.2840x over previous
import functools

import jax
import jax.numpy as jnp
from jax import lax
from jax.experimental import pallas as pl
from jax.experimental.pallas import tpu as pltpu
from jax.experimental.pallas import tpu_sc as plsc

_D = 768
_MAX_REL = 32
_S = 512
_AROWS = 144

_SHARED_ROUTE = {
    0: {5: "l", 7: "l", 9: "l", 11: "l", 13: "l", 15: "l"},
    1: {0: "h", 2: "h", 9: "l", 11: "l", 13: "l", 15: "l"},
    2: {0: "h", 2: "h", 4: "h", 6: "h", 13: "l", 15: "l"},
    3: {0: "h", 2: "h", 4: "h", 6: "h", 8: "h", 10: "h"},
}


def _clip_idx(p):
    return jnp.clip(511 - p, -_MAX_REL, _MAX_REL) + _MAX_REL


def _rpe_sc_kernel(table_hbm, out_hbm, idx_v, arena_v, c64_sh, c0_sh,
                   gsem, osem):
    nc = 2
    cid = lax.axis_index("c")
    sid = lax.axis_index("s")
    lane = lax.iota(jnp.int32, 16)
    wid = sid * nc + cid
    r = wid & 7
    q = lax.shift_right_logical(wid, 3)

    for u in range(_AROWS // 16):
        idx_v[pl.ds(0, 16)] = _clip_idx(440 + r + 16 * u + lane)
        pltpu.async_copy(
            table_hbm.at[idx_v], arena_v.at[pl.ds(16 * u, 16)], gsem
        ).wait()

    @pl.when(sid == 0)
    def _():
        for v in range(4):
            pltpu.sync_copy(
                arena_v.at[pl.ds(0, 32)], c64_sh.at[pl.ds(32 * v, 32)]
            )

    @pl.when(sid == 1)
    def _():
        for v in range(4):
            pltpu.sync_copy(
                arena_v.at[pl.ds(104, 32)], c0_sh.at[pl.ds(32 * v, 32)]
            )

    plsc.subcore_barrier()

    def fire(i, t, cq):
        s = 511 - i
        a = lax.shift_right_arithmetic(s, 3)
        for e in range(16):
            dst = out_hbm.at[i, pl.ds(32 * e, 32)]
            route = _SHARED_ROUTE[cq].get(e)
            if route is None:
                w0 = s + 32 * e
                g = jnp.where(
                    w0 >= 544, 13, jnp.where(w0 <= 448, 0, a + 4 * e - 55)
                )
                src = arena_v.at[pl.ds(8 * g, 32)]
            else:
                v = (wid + t + e) & 3
                blk = c64_sh if route == "h" else c0_sh
                src = blk.at[pl.ds(32 * v, 32)]
            pltpu.async_copy(src, dst, osem)

    def drain_slab(i):
        pltpu.make_async_copy(out_hbm.at[i], out_hbm.at[i], osem).wait()

    for cq in range(4):
        @pl.when(q == cq)
        def _(cq=cq):
            def slab_i(t):
                return 128 * cq + 8 * t + 7 - r

            def body(t, carry):
                fire(slab_i(t), t, cq)

                @pl.when(t >= 2)
                def _():
                    drain_slab(slab_i(t - 2))

                return carry

            lax.fori_loop(0, 16, body, 0)
            drain_slab(slab_i(14))
            drain_slab(slab_i(15))


def kernel(table, seq_len):
    del seq_len
    mesh = plsc.VectorSubcoreMesh(core_axis_name="c", subcore_axis_name="s")
    k = functools.partial(
        pl.kernel,
        mesh=mesh,
        out_type=jax.ShapeDtypeStruct((_S, _S, _D), jnp.float32),
        scratch_types=[
            pltpu.VMEM((16,), jnp.int32),
            pltpu.VMEM((_AROWS, _D), jnp.float32),
            pltpu.VMEM_SHARED((128, _D), jnp.float32),
            pltpu.VMEM_SHARED((128, _D), jnp.float32),
            pltpu.SemaphoreType.DMA,
            pltpu.SemaphoreType.DMA,
        ],
    )(_rpe_sc_kernel)
    return k(table)


# speedup vs baseline: 1.4993x; 1.0263x over previous
import functools

import jax
import jax.numpy as jnp
from jax import lax
from jax.experimental import pallas as pl
from jax.experimental.pallas import tpu as pltpu
from jax.experimental.pallas import tpu_sc as plsc

_D = 768
_MAX_REL = 32
_S = 512
_AROWS = 144

_SHARED_ROUTE = {
    0: {5: "l", 6: "l", 7: "l", 9: "l", 11: "l", 13: "l", 14: "l", 15: "l"},
    1: {0: "h", 1: "h", 2: "h", 9: "l", 10: "l", 11: "l", 13: "l", 15: "l"},
    2: {0: "h", 1: "h", 2: "h", 4: "h", 6: "h", 13: "l", 14: "l", 15: "l"},
    3: {0: "h", 1: "h", 2: "h", 3: "h", 4: "h", 6: "h", 8: "h", 10: "h"},
}


def _clip_idx(p):
    return jnp.clip(511 - p, -_MAX_REL, _MAX_REL) + _MAX_REL


def _rpe_sc_kernel(table_hbm, out_hbm, idx_v, arena_v, c64_sh, c0_sh,
                   gsem, osem):
    nc = 2
    cid = lax.axis_index("c")
    sid = lax.axis_index("s")
    lane = lax.iota(jnp.int32, 16)
    wid = sid * nc + cid
    r = wid & 7
    q = lax.shift_right_logical(wid, 3)

    for u in range(_AROWS // 16):
        idx_v[pl.ds(0, 16)] = _clip_idx(440 + r + 16 * u + lane)
        pltpu.async_copy(
            table_hbm.at[idx_v], arena_v.at[pl.ds(16 * u, 16)], gsem
        ).wait()

    @pl.when(sid == 0)
    def _():
        for v in range(4):
            pltpu.sync_copy(
                arena_v.at[pl.ds(0, 32)], c64_sh.at[pl.ds(32 * v, 32)]
            )

    @pl.when(sid == 1)
    def _():
        for v in range(4):
            pltpu.sync_copy(
                arena_v.at[pl.ds(104, 32)], c0_sh.at[pl.ds(32 * v, 32)]
            )

    plsc.subcore_barrier()

    def fire(i, t, cq):
        s = 511 - i
        a = lax.shift_right_arithmetic(s, 3)
        for e in range(16):
            dst = out_hbm.at[i, pl.ds(32 * e, 32)]
            route = _SHARED_ROUTE[cq].get(e)
            if route is None:
                w0 = s + 32 * e
                g = jnp.where(
                    w0 >= 544, 13, jnp.where(w0 <= 448, 0, a + 4 * e - 55)
                )
                src = arena_v.at[pl.ds(8 * g, 32)]
            else:
                v = (wid + t + e) & 3
                blk = c64_sh if route == "h" else c0_sh
                src = blk.at[pl.ds(32 * v, 32)]
            pltpu.async_copy(src, dst, osem)

    def drain_slab(i):
        pltpu.make_async_copy(out_hbm.at[i], out_hbm.at[i], osem).wait()

    for cq in range(4):
        @pl.when(q == cq)
        def _(cq=cq):
            def slab_i(t):
                return 128 * cq + 8 * t + 7 - r

            def body(t, carry):
                fire(slab_i(t), t, cq)

                @pl.when(t >= 2)
                def _():
                    drain_slab(slab_i(t - 2))

                return carry

            lax.fori_loop(0, 16, body, 0)
            drain_slab(slab_i(14))
            drain_slab(slab_i(15))


def kernel(table, seq_len):
    del seq_len
    mesh = plsc.VectorSubcoreMesh(core_axis_name="c", subcore_axis_name="s")
    k = functools.partial(
        pl.kernel,
        mesh=mesh,
        out_type=jax.ShapeDtypeStruct((_S, _S, _D), jnp.float32),
        scratch_types=[
            pltpu.VMEM((16,), jnp.int32),
            pltpu.VMEM((_AROWS, _D), jnp.float32),
            pltpu.VMEM_SHARED((128, _D), jnp.float32),
            pltpu.VMEM_SHARED((128, _D), jnp.float32),
            pltpu.SemaphoreType.DMA,
            pltpu.SemaphoreType.DMA,
        ],
    )(_rpe_sc_kernel)
    return k(table)
